# Initial kernel scaffold; baseline (speedup 1.0000x reference)
import jax
import jax.numpy as jnp
from jax.experimental import pallas as pl


def kernel(features, edge_index, lin1, att_src1, att_dst1, lin2, att_src3, att_dst3):
    raise NotImplementedError("write your pallas kernel here")


# speedup vs baseline: 26.9576x; 26.9576x over previous
import dataclasses

import jax
import jax.numpy as jnp
from jax import lax
from jax.experimental import pallas as pl
from jax.experimental.pallas import tpu as pltpu
from jax.experimental.pallas import tpu_sc as plsc

_N = 10000
_E = 320000
_NC, _NS = 2, 16
_NW = _NC * _NS
_EPT = _E // _NW
_CH = 80
_NCHUNK = _EPT // _CH
_NP = 10240
_RPT = _NP // _NS
_F32 = jnp.float32


def _leaky(x):
    return jnp.maximum(x, 0.2 * x)


def _elu(x):
    return jnp.where(x > 0, x, jnp.exp(jnp.minimum(x, 0.0)) - 1.0)


def _sc_w_body(src_hbm, dst_hbm, al_hbm, m_hbm,
               w1q_hbm, w2q_hbm, sp_hbm,
               a1, b1, a2, b2, m_v, src_v, dst_v, wo1_v, wo2_v,
               s1p, s2p, sem):
    cid = lax.axis_index("c")
    sid = lax.axis_index("s")
    wid = cid * _NS + sid
    ebase = wid * _EPT

    pltpu.sync_copy(al_hbm.at[pl.ds(0, _N)], a1)
    pltpu.sync_copy(al_hbm.at[pl.ds(_N, _N)], b1)
    pltpu.sync_copy(al_hbm.at[pl.ds(2 * _N, _N)], a2)
    pltpu.sync_copy(al_hbm.at[pl.ds(3 * _N, _N)], b2)
    pltpu.sync_copy(m_hbm, m_v)

    zf = jnp.zeros((16,), _F32)

    @pl.loop(0, _NP, step=16)
    def _zero_s(r):
        s1p[pl.ds(r, 16)] = zf
        s2p[pl.ds(r, 16)] = zf

    m1 = m_v[pl.ds(0, 16)]
    m2 = m_v[pl.ds(16, 16)]

    @pl.loop(0, _NCHUNK)
    def _chunk(j):
        cb = ebase + j * _CH
        pltpu.sync_copy(src_hbm.at[pl.ds(cb, _CH)], src_v)
        pltpu.sync_copy(dst_hbm.at[pl.ds(cb, _CH)], dst_v)

        @pl.loop(0, _CH, step=16)
        def _w(i):
            s16 = src_v[pl.ds(i, 16)]
            d16 = dst_v[pl.ds(i, 16)]
            l1 = plsc.load_gather(a1, [s16]) + plsc.load_gather(b1, [d16])
            w1 = jnp.exp(_leaky(l1) - m1)
            l2 = plsc.load_gather(a2, [s16]) + plsc.load_gather(b2, [d16])
            w2 = jnp.exp(_leaky(l2) - m2)
            wo1_v[pl.ds(i, 16)] = w1
            wo2_v[pl.ds(i, 16)] = w2
            plsc.addupdate_scatter(s1p, [d16], w1)
            plsc.addupdate_scatter(s2p, [d16], w2)

        pltpu.sync_copy(wo1_v, w1q_hbm.at[pl.ds(cb, _CH)])
        pltpu.sync_copy(wo2_v, w2q_hbm.at[pl.ds(cb, _CH)])

    base = wid * 2 * _NP
    pltpu.sync_copy(s1p, sp_hbm.at[pl.ds(base, _NP)])
    pltpu.sync_copy(s2p, sp_hbm.at[pl.ds(base + _NP, _NP)])


def _sc_row_body(xs_hbm, src_hbm, dst_hbm, w1q_hbm, w2q_hbm,
                 acc_hbm,
                 src_v, dst_v, w1_v, w2_v, rows, acc_s, sem):
    cid = lax.axis_index("c")
    sid = lax.axis_index("s")
    wid = cid * _NS + sid
    ebase = wid * _EPT

    zf = jnp.zeros((16,), _F32)

    @pl.loop(0, _CH)
    def _zero_bufs(r):
        for c in range(8):
            rows[r, pl.ds(c * 16, 16)] = zf

    r0 = sid * _RPT
    for k in range(_RPT // _CH):
        pltpu.sync_copy(rows, acc_s.at[pl.ds(r0 + k * _CH, _CH)])
    plsc.subcore_barrier()

    @pl.loop(0, _NCHUNK)
    def _chunk(j):
        cb = ebase + j * _CH
        pltpu.sync_copy(src_hbm.at[pl.ds(cb, _CH)], src_v)
        pltpu.sync_copy(dst_hbm.at[pl.ds(cb, _CH)], dst_v)
        pltpu.sync_copy(w1q_hbm.at[pl.ds(cb, _CH)], w1_v)
        pltpu.sync_copy(w2q_hbm.at[pl.ds(cb, _CH)], w2_v)
        pltpu.async_copy(xs_hbm.at[src_v], rows, sem).wait()

        @pl.loop(0, _CH)
        def _scale(r):
            rr = jnp.full((16,), r, jnp.int32)
            w1b = plsc.load_gather(w1_v, [rr])
            w2b = plsc.load_gather(w2_v, [rr])
            for c in range(4):
                sl = pl.ds(c * 16, 16)
                rows[r, sl] = rows[r, sl] * w1b
            for c in range(4, 8):
                sl = pl.ds(c * 16, 16)
                rows[r, sl] = rows[r, sl] * w2b

        pltpu.sync_copy(rows, acc_s.at[dst_v], add=True)

    plsc.subcore_barrier()
    pltpu.sync_copy(acc_s.at[pl.ds(r0, _RPT)],
                    acc_hbm.at[pl.ds(cid * _NP + r0, _RPT)])


def _sc_compiler_params():
    cp = pltpu.CompilerParams()
    if "needs_layout_passes" in pltpu.CompilerParams.__dataclass_fields__:
        cp = dataclasses.replace(cp, needs_layout_passes=False)
    return cp


def _sc_mesh():
    return plsc.VectorSubcoreMesh(core_axis_name="c", subcore_axis_name="s")


def _sc_attn(xs, src, dst, alph, mvec):
    alph = alph.reshape(-1)
    w_kern = pl.kernel(
        _sc_w_body,
        out_type=[jax.ShapeDtypeStruct((_E,), _F32),
                  jax.ShapeDtypeStruct((_E,), _F32),
                  jax.ShapeDtypeStruct((_NW * 2 * _NP,), _F32)],
        mesh=_sc_mesh(),
        scratch_types=[
            pltpu.VMEM((_N,), _F32),
            pltpu.VMEM((_N,), _F32),
            pltpu.VMEM((_N,), _F32),
            pltpu.VMEM((_N,), _F32),
            pltpu.VMEM((32,), _F32),
            pltpu.VMEM((_CH,), jnp.int32),
            pltpu.VMEM((_CH,), jnp.int32),
            pltpu.VMEM((_CH,), _F32),
            pltpu.VMEM((_CH,), _F32),
            pltpu.VMEM((_NP,), _F32),
            pltpu.VMEM((_NP,), _F32),
            pltpu.SemaphoreType.DMA,
        ],
        compiler_params=_sc_compiler_params(),
    )
    w1q, w2q, sp = w_kern(src, dst, alph, mvec)
    sp = sp.reshape(_NW, 2, _NP)

    row_kern = pl.kernel(
        _sc_row_body,
        out_type=[jax.ShapeDtypeStruct((_NC * _NP, 128), _F32)],
        mesh=_sc_mesh(),
        scratch_types=[
            pltpu.VMEM((_CH,), jnp.int32),
            pltpu.VMEM((_CH,), jnp.int32),
            pltpu.VMEM((_CH,), _F32),
            pltpu.VMEM((_CH,), _F32),
            pltpu.VMEM((_CH, 128), _F32),
            pltpu.VMEM_SHARED((_NP, 128), _F32),
            pltpu.SemaphoreType.DMA,
        ],
        compiler_params=_sc_compiler_params(),
    )
    (acc,) = row_kern(xs, src, dst, w1q, w2q)
    return acc.reshape(_NC, _NP, 128), sp


def _attn_terms(xs, att_s, att_d):
    a = jnp.sum(xs * att_s, axis=1)
    b = jnp.sum(xs * att_d, axis=1)
    m = _leaky(jnp.max(a) + jnp.max(b))
    return a, b, m


def _tc_prep_body(x1_r, x2_r, lin_r, as_r, ad_r, xs_o, al_o, m_o):
    lin = lin_r[...]
    xs1 = jnp.dot(x1_r[...], lin, preferred_element_type=_F32)
    xs2 = jnp.dot(x2_r[...], lin, preferred_element_type=_F32)
    a1, b1, m1 = _attn_terms(xs1, as_r[...], ad_r[...])
    a2, b2, m2 = _attn_terms(xs2, as_r[...], ad_r[...])
    xs_o[...] = jnp.concatenate([xs1, xs2], axis=1)
    al_o[...] = jnp.concatenate(
        [a1[None, :], b1[None, :], a2[None, :], b2[None, :]], axis=0)
    m_o[...] = jnp.concatenate([jnp.full((16,), m1), jnp.full((16,), m2)])


def _tc_prep(x1, x2, lin, att_s, att_d):
    return pl.pallas_call(
        _tc_prep_body,
        out_shape=[jax.ShapeDtypeStruct((_N, 128), _F32),
                   jax.ShapeDtypeStruct((4, _N), _F32),
                   jax.ShapeDtypeStruct((32,), _F32)],
    )(x1, x2, lin, att_s, att_d)


def _combine(acc_r, sp_r):
    a = acc_r[...].sum(axis=0)[:_N]
    s = sp_r[...].sum(axis=0)
    h = _elu(a[:, :64] / (s[0][:_N][:, None] + 1e-16))
    rh = _elu(a[:, 64:] / (s[1][:_N][:, None] + 1e-16))
    return h, rh


def _tc_mid_body(acc_r, sac_r, lin2_r, as_r, ad_r,
                 h2_o, rh2_o, xs_o, al_o, m_o, sum_o):
    h1, rh1 = _combine(acc_r, sac_r)
    lin2 = lin2_r[...]
    h2 = jnp.dot(h1, lin2, preferred_element_type=_F32)
    rh2 = jnp.dot(rh1, lin2, preferred_element_type=_F32)
    h2_o[...] = h2
    rh2_o[...] = rh2
    dn = (((1,), (1,)), ((), ()))
    xs3 = lax.dot_general(h2, lin2, dn, preferred_element_type=_F32)
    xs4 = lax.dot_general(rh2, lin2, dn, preferred_element_type=_F32)
    a3, b3, m3 = _attn_terms(xs3, as_r[...], ad_r[...])
    a4, b4, m4 = _attn_terms(xs4, as_r[...], ad_r[...])
    xs_o[...] = jnp.concatenate([xs3, xs4], axis=1)
    al_o[...] = jnp.concatenate(
        [a3[None, :], b3[None, :], a4[None, :], b4[None, :]], axis=0)
    m_o[...] = jnp.concatenate([jnp.full((16,), m3), jnp.full((16,), m4)])
    sum_o[...] = jax.nn.sigmoid(jnp.mean(h2, axis=0))


def _tc_mid(acc, sac, lin2, att_s, att_d):
    return pl.pallas_call(
        _tc_mid_body,
        out_shape=[jax.ShapeDtypeStruct((_N, 32), _F32),
                   jax.ShapeDtypeStruct((_N, 32), _F32),
                   jax.ShapeDtypeStruct((_N, 128), _F32),
                   jax.ShapeDtypeStruct((4, _N), _F32),
                   jax.ShapeDtypeStruct((32,), _F32),
                   jax.ShapeDtypeStruct((32,), _F32)],
    )(acc, sac, lin2, att_s, att_d)


def _tc_fin_body(acc_r, sac_r, lin1_r, h4_o, rh4_o):
    h3, rh3 = _combine(acc_r, sac_r)
    lin1 = lin1_r[...]
    dn = (((1,), (1,)), ((), ()))
    h4_o[...] = lax.dot_general(h3, lin1, dn, preferred_element_type=_F32)
    rh4_o[...] = lax.dot_general(rh3, lin1, dn, preferred_element_type=_F32)


def _tc_fin(acc, sac, lin1):
    return pl.pallas_call(
        _tc_fin_body,
        out_shape=[jax.ShapeDtypeStruct((_N, 128), _F32),
                   jax.ShapeDtypeStruct((_N, 128), _F32)],
    )(acc, sac, lin1)


def kernel(features, edge_index, lin1, att_src1, att_dst1,
           lin2, att_src3, att_dst3):
    perm = jax.random.permutation(jax.random.key(42), _N)
    randf = features[perm]
    src = edge_index[0]
    dst = edge_index[1]

    as1 = att_src1.reshape(1, -1)
    ad1 = att_dst1.reshape(1, -1)
    as3 = att_src3.reshape(1, -1)
    ad3 = att_dst3.reshape(1, -1)

    xs12, al12, m12 = _tc_prep(features, randf, lin1, as1, ad1)
    acc1, sac1 = _sc_attn(xs12, src, dst, al12, m12)
    h2, rh2, xs34, al34, m34, summ = _tc_mid(acc1, sac1, lin2, as3, ad3)
    acc2, sac2 = _sc_attn(xs34, src, dst, al34, m34)
    h4, rh4 = _tc_fin(acc2, sac2, lin1)
    return (h2, h4, h4, rh2, rh4, rh4, summ)


# speedup vs baseline: 40.7484x; 1.5116x over previous
import dataclasses

import jax
import jax.numpy as jnp
from jax import lax
from jax.experimental import pallas as pl
from jax.experimental.pallas import tpu as pltpu
from jax.experimental.pallas import tpu_sc as plsc

_N = 10000
_E = 320000
_NC, _NS = 2, 16
_NW = _NC * _NS
_EPT = _E // _NW
_CH = 80
_NCHUNK = _EPT // _CH
_NP = 10240
_RPT = _NP // _NS
_F32 = jnp.float32


def _leaky(x):
    return jnp.maximum(x, 0.2 * x)


def _elu(x):
    return jnp.where(x > 0, x, jnp.exp(jnp.minimum(x, 0.0)) - 1.0)


def _sc_w_body(src_hbm, dst_hbm, al_hbm, m_hbm,
               w1q_hbm, w2q_hbm, sp_hbm,
               a1, b1, a2, b2, m_v, src_v, dst_v, wo1_v, wo2_v,
               s1p, s2p, sem):
    cid = lax.axis_index("c")
    sid = lax.axis_index("s")
    wid = cid * _NS + sid
    ebase = wid * _EPT

    pltpu.sync_copy(al_hbm.at[pl.ds(0, _N)], a1)
    pltpu.sync_copy(al_hbm.at[pl.ds(_N, _N)], b1)
    pltpu.sync_copy(al_hbm.at[pl.ds(2 * _N, _N)], a2)
    pltpu.sync_copy(al_hbm.at[pl.ds(3 * _N, _N)], b2)
    pltpu.sync_copy(m_hbm, m_v)

    zf = jnp.zeros((16,), _F32)

    @pl.loop(0, _NP, step=16)
    def _zero_s(r):
        s1p[pl.ds(r, 16)] = zf
        s2p[pl.ds(r, 16)] = zf

    m1 = m_v[pl.ds(0, 16)]
    m2 = m_v[pl.ds(16, 16)]

    @pl.loop(0, _NCHUNK)
    def _chunk(j):
        cb = ebase + j * _CH
        f1 = pltpu.async_copy(src_hbm.at[pl.ds(cb, _CH)], src_v, sem)
        f2 = pltpu.async_copy(dst_hbm.at[pl.ds(cb, _CH)], dst_v, sem)
        f1.wait()
        f2.wait()

        @pl.loop(0, _CH, step=16)
        def _w(i):
            s16 = src_v[pl.ds(i, 16)]
            d16 = dst_v[pl.ds(i, 16)]
            l1 = plsc.load_gather(a1, [s16]) + plsc.load_gather(b1, [d16])
            w1 = jnp.exp(_leaky(l1) - m1)
            l2 = plsc.load_gather(a2, [s16]) + plsc.load_gather(b2, [d16])
            w2 = jnp.exp(_leaky(l2) - m2)
            wo1_v[pl.ds(i, 16)] = w1
            wo2_v[pl.ds(i, 16)] = w2
            plsc.addupdate_scatter(s1p, [d16], w1)
            plsc.addupdate_scatter(s2p, [d16], w2)

        pltpu.sync_copy(wo1_v, w1q_hbm.at[pl.ds(cb, _CH)])
        pltpu.sync_copy(wo2_v, w2q_hbm.at[pl.ds(cb, _CH)])

    base = wid * 2 * _NP
    pltpu.sync_copy(s1p, sp_hbm.at[pl.ds(base, _NP)])
    pltpu.sync_copy(s2p, sp_hbm.at[pl.ds(base + _NP, _NP)])


def _sc_row_body(xs_hbm, src_hbm, dst_hbm, w1q_hbm, w2q_hbm,
                 acc_hbm,
                 src_v, dst_v, w1_v, w2_v, rows, acc_s, sem):
    cid = lax.axis_index("c")
    sid = lax.axis_index("s")
    wid = cid * _NS + sid
    ebase = wid * _EPT

    zf = jnp.zeros((16,), _F32)

    @pl.loop(0, _CH)
    def _zero_bufs(r):
        for c in range(8):
            rows[r, pl.ds(c * 16, 16)] = zf

    r0 = sid * _RPT
    for k in range(_RPT // _CH):
        pltpu.sync_copy(rows, acc_s.at[pl.ds(r0 + k * _CH, _CH)])
    plsc.subcore_barrier()

    @pl.loop(0, _NCHUNK)
    def _chunk(j):
        cb = ebase + j * _CH
        f1 = pltpu.async_copy(src_hbm.at[pl.ds(cb, _CH)], src_v, sem)
        f2 = pltpu.async_copy(dst_hbm.at[pl.ds(cb, _CH)], dst_v, sem)
        f3 = pltpu.async_copy(w1q_hbm.at[pl.ds(cb, _CH)], w1_v, sem)
        f4 = pltpu.async_copy(w2q_hbm.at[pl.ds(cb, _CH)], w2_v, sem)
        f1.wait()
        f2.wait()
        f3.wait()
        f4.wait()
        pltpu.async_copy(xs_hbm.at[src_v], rows, sem).wait()

        @plsc.parallel_loop(0, _CH, unroll=4)
        def _scale(r):
            rr = jnp.full((16,), r, jnp.int32)
            w1b = plsc.load_gather(w1_v, [rr])
            w2b = plsc.load_gather(w2_v, [rr])
            for c in range(4):
                sl = pl.ds(c * 16, 16)
                rows[r, sl] = rows[r, sl] * w1b
            for c in range(4, 8):
                sl = pl.ds(c * 16, 16)
                rows[r, sl] = rows[r, sl] * w2b

        pltpu.sync_copy(rows, acc_s.at[dst_v], add=True)

    plsc.subcore_barrier()
    pltpu.sync_copy(acc_s.at[pl.ds(r0, _RPT)],
                    acc_hbm.at[pl.ds(cid * _NP + r0, _RPT)])


def _sc_compiler_params():
    cp = pltpu.CompilerParams()
    if "needs_layout_passes" in pltpu.CompilerParams.__dataclass_fields__:
        cp = dataclasses.replace(cp, needs_layout_passes=False)
    return cp


def _sc_mesh():
    return plsc.VectorSubcoreMesh(core_axis_name="c", subcore_axis_name="s")


def _sc_attn(xs, src, dst, alph, mvec):
    alph = alph.reshape(-1)
    w_kern = pl.kernel(
        _sc_w_body,
        out_type=[jax.ShapeDtypeStruct((_E,), _F32),
                  jax.ShapeDtypeStruct((_E,), _F32),
                  jax.ShapeDtypeStruct((_NW * 2 * _NP,), _F32)],
        mesh=_sc_mesh(),
        scratch_types=[
            pltpu.VMEM((_N,), _F32),
            pltpu.VMEM((_N,), _F32),
            pltpu.VMEM((_N,), _F32),
            pltpu.VMEM((_N,), _F32),
            pltpu.VMEM((32,), _F32),
            pltpu.VMEM((_CH,), jnp.int32),
            pltpu.VMEM((_CH,), jnp.int32),
            pltpu.VMEM((_CH,), _F32),
            pltpu.VMEM((_CH,), _F32),
            pltpu.VMEM((_NP,), _F32),
            pltpu.VMEM((_NP,), _F32),
            pltpu.SemaphoreType.DMA,
        ],
        compiler_params=_sc_compiler_params(),
    )
    w1q, w2q, sp = w_kern(src, dst, alph, mvec)
    sp = sp.reshape(_NW, 2, _NP)

    row_kern = pl.kernel(
        _sc_row_body,
        out_type=[jax.ShapeDtypeStruct((_NC * _NP, 128), _F32)],
        mesh=_sc_mesh(),
        scratch_types=[
            pltpu.VMEM((_CH,), jnp.int32),
            pltpu.VMEM((_CH,), jnp.int32),
            pltpu.VMEM((_CH,), _F32),
            pltpu.VMEM((_CH,), _F32),
            pltpu.VMEM((_CH, 128), _F32),
            pltpu.VMEM_SHARED((_NP, 128), _F32),
            pltpu.SemaphoreType.DMA,
        ],
        compiler_params=_sc_compiler_params(),
    )
    (acc,) = row_kern(xs, src, dst, w1q, w2q)
    return acc.reshape(_NC, _NP, 128), sp


def _attn_terms(xs, att_s, att_d):
    a = jnp.sum(xs * att_s, axis=1)
    b = jnp.sum(xs * att_d, axis=1)
    m = _leaky(jnp.max(a) + jnp.max(b))
    return a, b, m


def _tc_prep_body(x1_r, x2_r, lin_r, as_r, ad_r, xs_o, al_o, m_o):
    lin = lin_r[...]
    xs1 = jnp.dot(x1_r[...], lin, preferred_element_type=_F32)
    xs2 = jnp.dot(x2_r[...], lin, preferred_element_type=_F32)
    a1, b1, m1 = _attn_terms(xs1, as_r[...], ad_r[...])
    a2, b2, m2 = _attn_terms(xs2, as_r[...], ad_r[...])
    xs_o[...] = jnp.concatenate([xs1, xs2], axis=1)
    al_o[...] = jnp.concatenate(
        [a1[None, :], b1[None, :], a2[None, :], b2[None, :]], axis=0)
    m_o[...] = jnp.concatenate([jnp.full((16,), m1), jnp.full((16,), m2)])


def _tc_prep(x1, x2, lin, att_s, att_d):
    return pl.pallas_call(
        _tc_prep_body,
        out_shape=[jax.ShapeDtypeStruct((_N, 128), _F32),
                   jax.ShapeDtypeStruct((4, _N), _F32),
                   jax.ShapeDtypeStruct((32,), _F32)],
    )(x1, x2, lin, att_s, att_d)


def _combine(acc_r, sp_r):
    a = acc_r[...].sum(axis=0)[:_N]
    s = sp_r[...].sum(axis=0)
    h = _elu(a[:, :64] / (s[0][:_N][:, None] + 1e-16))
    rh = _elu(a[:, 64:] / (s[1][:_N][:, None] + 1e-16))
    return h, rh


def _tc_mid_body(acc_r, sac_r, lin2_r, as_r, ad_r,
                 h2_o, rh2_o, xs_o, al_o, m_o, sum_o):
    h1, rh1 = _combine(acc_r, sac_r)
    lin2 = lin2_r[...]
    h2 = jnp.dot(h1, lin2, preferred_element_type=_F32)
    rh2 = jnp.dot(rh1, lin2, preferred_element_type=_F32)
    h2_o[...] = h2
    rh2_o[...] = rh2
    dn = (((1,), (1,)), ((), ()))
    xs3 = lax.dot_general(h2, lin2, dn, preferred_element_type=_F32)
    xs4 = lax.dot_general(rh2, lin2, dn, preferred_element_type=_F32)
    a3, b3, m3 = _attn_terms(xs3, as_r[...], ad_r[...])
    a4, b4, m4 = _attn_terms(xs4, as_r[...], ad_r[...])
    xs_o[...] = jnp.concatenate([xs3, xs4], axis=1)
    al_o[...] = jnp.concatenate(
        [a3[None, :], b3[None, :], a4[None, :], b4[None, :]], axis=0)
    m_o[...] = jnp.concatenate([jnp.full((16,), m3), jnp.full((16,), m4)])
    sum_o[...] = jax.nn.sigmoid(jnp.mean(h2, axis=0))


def _tc_mid(acc, sac, lin2, att_s, att_d):
    return pl.pallas_call(
        _tc_mid_body,
        out_shape=[jax.ShapeDtypeStruct((_N, 32), _F32),
                   jax.ShapeDtypeStruct((_N, 32), _F32),
                   jax.ShapeDtypeStruct((_N, 128), _F32),
                   jax.ShapeDtypeStruct((4, _N), _F32),
                   jax.ShapeDtypeStruct((32,), _F32),
                   jax.ShapeDtypeStruct((32,), _F32)],
    )(acc, sac, lin2, att_s, att_d)


def _tc_fin_body(acc_r, sac_r, lin1_r, h4_o, rh4_o):
    h3, rh3 = _combine(acc_r, sac_r)
    lin1 = lin1_r[...]
    dn = (((1,), (1,)), ((), ()))
    h4_o[...] = lax.dot_general(h3, lin1, dn, preferred_element_type=_F32)
    rh4_o[...] = lax.dot_general(rh3, lin1, dn, preferred_element_type=_F32)


def _tc_fin(acc, sac, lin1):
    return pl.pallas_call(
        _tc_fin_body,
        out_shape=[jax.ShapeDtypeStruct((_N, 128), _F32),
                   jax.ShapeDtypeStruct((_N, 128), _F32)],
    )(acc, sac, lin1)


def kernel(features, edge_index, lin1, att_src1, att_dst1,
           lin2, att_src3, att_dst3):
    perm = jax.random.permutation(jax.random.key(42), _N)
    randf = features[perm]
    src = edge_index[0]
    dst = edge_index[1]

    as1 = att_src1.reshape(1, -1)
    ad1 = att_dst1.reshape(1, -1)
    as3 = att_src3.reshape(1, -1)
    ad3 = att_dst3.reshape(1, -1)

    xs12, al12, m12 = _tc_prep(features, randf, lin1, as1, ad1)
    acc1, sac1 = _sc_attn(xs12, src, dst, al12, m12)
    h2, rh2, xs34, al34, m34, summ = _tc_mid(acc1, sac1, lin2, as3, ad3)
    acc2, sac2 = _sc_attn(xs34, src, dst, al34, m34)
    h4, rh4 = _tc_fin(acc2, sac2, lin1)
    return (h2, h4, h4, rh2, rh4, rh4, summ)
